# baseline (speedup 1.0000x reference)
import functools

import jax
import jax.numpy as jnp
from jax import lax
from jax.experimental import pallas as pl
from jax.experimental.pallas import tpu as pltpu
from jax.experimental.pallas import tpu_sc as plsc

C_IN, C_OUT, H, W = 96, 96, 224, 224
HW = H * W
K = 9
TH = 8
NT = H // TH
HP = 232
NC, NS = 2, 16
NW = NC * NS
GROUPS = K * H
PER_W = GROUPS // NW


def _proj_body(x_ref, wq_ref, bq_ref, wk_ref, bk_ref, q_ref, k_ref, xp_ref):
    x = x_ref[...]
    q_ref[...] = jnp.dot(wq_ref[...], x, preferred_element_type=jnp.float32) + bq_ref[...]
    k_ref[...] = jnp.dot(wk_ref[...], x, preferred_element_type=jnp.float32) + bk_ref[...]
    xt = x.T
    xp_ref[...] = jnp.concatenate(
        [xt, jnp.zeros((HW // NT, 128 - C_IN), jnp.float32)], axis=1)


def _proj(x_cm, Wq, bq, Wk, bk):
    blk = HW // NT
    return pl.pallas_call(
        _proj_body,
        grid=(NT,),
        in_specs=[
            pl.BlockSpec((C_IN, blk), lambda i: (0, i)),
            pl.BlockSpec((32, C_IN), lambda i: (0, 0)),
            pl.BlockSpec((32, 1), lambda i: (0, 0)),
            pl.BlockSpec((32, C_IN), lambda i: (0, 0)),
            pl.BlockSpec((32, 1), lambda i: (0, 0)),
        ],
        out_specs=[
            pl.BlockSpec((32, blk), lambda i: (0, i)),
            pl.BlockSpec((32, blk), lambda i: (0, i)),
            pl.BlockSpec((blk, 128), lambda i: (i, 0)),
        ],
        out_shape=[
            jax.ShapeDtypeStruct((32, HW), jnp.float32),
            jax.ShapeDtypeStruct((32, HW), jnp.float32),
            jax.ShapeDtypeStruct((HW, 128), jnp.float32),
        ],
    )(x_cm, Wq, bq, Wk, bk)


def _search_body(q_ref, ka_ref, kb_ref, wts_ref, idx_ref):
    i = pl.program_id(0)
    q = q_ref[...]
    kc = jnp.concatenate([ka_ref[...], kb_ref[...]], axis=1)
    ds = []
    for o in range(81):
        dy, dx = o // 9 - 4, o % 9 - 4
        ks = kc[:, dy + 4:dy + 12, dx + 4:dx + 228]
        ds.append(jnp.sum(q * ks, axis=0))
    D = jnp.stack(ds, axis=0)
    lane0 = lax.broadcasted_iota(jnp.int32, (81, TH, W), 0)
    rem = D
    tds, tis = [], []
    for _ in range(K):
        m = jnp.max(rem, axis=0, keepdims=True)
        eq = rem == m
        first = jnp.min(jnp.where(eq, lane0, 81), axis=0, keepdims=True)
        tds.append(m[0])
        tis.append(first[0])
        rem = jnp.where(lane0 == first, -jnp.inf, rem)
    topd = jnp.stack(tds, axis=0)
    topi = jnp.stack(tis, axis=0)
    s = topd * 0.1
    e = jnp.exp(s - s[0:1])
    wts_ref[...] = e / jnp.sum(e, axis=0, keepdims=True)
    dyk = topi // 9 - 4
    dxk = topi % 9 - 4
    hloc = lax.broadcasted_iota(jnp.int32, (K, TH, W), 1) + i * TH
    wloc = lax.broadcasted_iota(jnp.int32, (K, TH, W), 2)
    gy = jnp.clip(hloc + dyk, 0, H - 1)
    gx = jnp.clip(wloc + dxk, 0, W - 1)
    idx_ref[...] = gy * W + gx


def _search(q3, kfp):
    return pl.pallas_call(
        _search_body,
        grid=(NT,),
        in_specs=[
            pl.BlockSpec((32, TH, W), lambda i: (0, i, 0)),
            pl.BlockSpec((32, TH, W + 8), lambda i: (0, i, 0)),
            pl.BlockSpec((32, TH, W + 8), lambda i: (0, i + 1, 0)),
        ],
        out_specs=[
            pl.BlockSpec((K, TH, W), lambda i: (0, i, 0)),
            pl.BlockSpec((K, TH, W), lambda i: (0, i, 0)),
        ],
        out_shape=[
            jax.ShapeDtypeStruct((K, H, W), jnp.float32),
            jax.ShapeDtypeStruct((K, H, W), jnp.int32),
        ],
    )(q3, kfp, kfp)


_sc_mesh = plsc.VectorSubcoreMesh(core_axis_name="c", subcore_axis_name="s")


@functools.partial(
    pl.kernel,
    mesh=_sc_mesh,
    out_type=jax.ShapeDtypeStruct((K * NT * 10 * W, 128), jnp.float32),
    scratch_types=[
        pltpu.VMEM((W,), jnp.int32),
        pltpu.VMEM((W, 128), jnp.float32),
        pltpu.SemaphoreType.DMA,
    ],
)
def _sc_gather(x_hbm, idx_hbm, vol_hbm, idx_v, rows_v, sem):
    wid = lax.axis_index("s") * NC + lax.axis_index("c")

    def zfill(j, carry):
        for c in range(8):
            rows_v[j, pl.ds(c * 16, 16)] = jnp.zeros((16,), jnp.float32)
        return carry

    lax.fori_loop(0, W, zfill, 0)

    @pl.when(wid < 2 * K)
    def _():
        k = wid // 2
        blk = jnp.where(wid % 2 == 0, k * NT * 10, (k * NT + NT - 1) * 10 + 9)
        pltpu.sync_copy(rows_v, vol_hbm.at[pl.ds(blk * W, W)])

    def body(j, carry):
        g = wid * PER_W + j
        k = g // H
        h = g % H
        r = h + 1
        i1 = r // 8
        o1 = r % 8
        pltpu.sync_copy(idx_hbm.at[pl.ds(g * W, W)], idx_v)
        pltpu.async_copy(x_hbm.at[idx_v], rows_v, sem).wait()

        @pl.when(i1 <= NT - 1)
        def _():
            dst = ((k * NT + i1) * 10 + o1) * W
            pltpu.sync_copy(rows_v, vol_hbm.at[pl.ds(dst, W)])

        @pl.when((o1 <= 1) & (i1 >= 1))
        def _():
            dst = ((k * NT + i1 - 1) * 10 + o1 + 8) * W
            pltpu.sync_copy(rows_v, vol_hbm.at[pl.ds(dst, W)])

        return carry

    lax.fori_loop(0, PER_W, body, 0)


def _conv_body(va_ref, wa_ref, wb_ref, wf_ref, bc_ref, out_ref):
    va = va_ref[...][:, 0]
    wc = jnp.concatenate([wa_ref[...], wb_ref[...]], axis=1)[:, :10]
    wvs = []
    for k in range(K):
        wk = wc[k][:, :, None]
        wvs.append((va[k, :, :, :C_IN] * wk).astype(jnp.bfloat16))
    bv = jnp.concatenate(wvs, axis=-1)
    zc = jnp.zeros((10, 1, K * C_IN), jnp.bfloat16)
    bvp = jnp.concatenate([zc, bv, zc], axis=1)
    acc = jnp.zeros((TH * W, C_OUT), jnp.float32) + bc_ref[...]
    for dy in range(3):
        for dx in range(3):
            patch = bvp[dy:dy + TH, dx:dx + W, :].reshape(TH * W, K * C_IN)
            acc = acc + jnp.dot(patch, wf_ref[dy * 3 + dx],
                                preferred_element_type=jnp.float32)
    out_ref[...] = acc.T.reshape(C_OUT, TH, W)


def _conv(vol, wtsp, Wf, bc):
    return pl.pallas_call(
        _conv_body,
        grid=(NT,),
        in_specs=[
            pl.BlockSpec((K, 1, 10, W, 128), lambda i: (0, i, 0, 0, 0)),
            pl.BlockSpec((K, TH, W), lambda i: (0, i, 0)),
            pl.BlockSpec((K, TH, W), lambda i: (0, i + 1, 0)),
            pl.BlockSpec((9, K * C_IN, C_OUT), lambda i: (0, 0, 0)),
            pl.BlockSpec((1, C_OUT), lambda i: (0, 0)),
        ],
        out_specs=pl.BlockSpec((C_OUT, TH, W), lambda i: (0, i, 0)),
        out_shape=jax.ShapeDtypeStruct((C_OUT, H, W), jnp.float32),
    )(vol, wtsp, wtsp, Wf, bc)


def kernel(x, fflow, bflow, Wq, bq, Wk, bk, Wc, bc):
    x_cm = x[0].reshape(C_IN, HW)

    q_cm, kf_cm, x_pix128 = _proj(x_cm, Wq, bq.reshape(32, 1), Wk, bk.reshape(32, 1))
    q3 = q_cm.reshape(32, H, W)
    kfp = jnp.pad(kf_cm.reshape(32, H, W), ((0, 0), (4, 4), (4, 4)), mode="edge")

    wts9, idx9 = _search(q3, kfp)

    vol = _sc_gather(x_pix128, idx9.reshape(-1))
    vol5 = vol.reshape(K, NT, 10, W, 128)

    wtsp = jnp.pad(wts9, ((0, 0), (1, HP - H - 1), (0, 0)))
    Wf = jnp.transpose(Wc, (3, 4, 2, 1, 0)).reshape(
        9, K * C_IN, C_OUT).astype(jnp.bfloat16)
    out_cm = _conv(vol5, wtsp, Wf, bc.reshape(1, C_OUT))

    return out_cm[None]


# speedup vs baseline: 1.0186x; 1.0186x over previous
import functools

import jax
import jax.numpy as jnp
from jax import lax
from jax.experimental import pallas as pl
from jax.experimental.pallas import tpu as pltpu
from jax.experimental.pallas import tpu_sc as plsc

C_IN, C_OUT, H, W = 96, 96, 224, 224
HW = H * W
K = 9
TH = 8
NT = H // TH
HP = 232
NC, NS = 2, 16
NW = NC * NS
GROUPS = K * H
PER_W = GROUPS // NW


def _proj_body(x_ref, wq_ref, bq_ref, wk_ref, bk_ref, q_ref, k_ref, xp_ref):
    x = x_ref[...]
    q_ref[...] = jnp.dot(wq_ref[...], x, preferred_element_type=jnp.float32) + bq_ref[...]
    k_ref[...] = jnp.dot(wk_ref[...], x, preferred_element_type=jnp.float32) + bk_ref[...]
    xt = x.T
    xp_ref[...] = jnp.concatenate(
        [xt, jnp.zeros((HW // NT, 128 - C_IN), jnp.float32)], axis=1)


def _proj(x_cm, Wq, bq, Wk, bk):
    blk = HW // NT
    return pl.pallas_call(
        _proj_body,
        grid=(NT,),
        in_specs=[
            pl.BlockSpec((C_IN, blk), lambda i: (0, i)),
            pl.BlockSpec((32, C_IN), lambda i: (0, 0)),
            pl.BlockSpec((32, 1), lambda i: (0, 0)),
            pl.BlockSpec((32, C_IN), lambda i: (0, 0)),
            pl.BlockSpec((32, 1), lambda i: (0, 0)),
        ],
        out_specs=[
            pl.BlockSpec((32, blk), lambda i: (0, i)),
            pl.BlockSpec((32, blk), lambda i: (0, i)),
            pl.BlockSpec((blk, 128), lambda i: (i, 0)),
        ],
        out_shape=[
            jax.ShapeDtypeStruct((32, HW), jnp.float32),
            jax.ShapeDtypeStruct((32, HW), jnp.float32),
            jax.ShapeDtypeStruct((HW, 128), jnp.float32),
        ],
    )(x_cm, Wq, bq, Wk, bk)


TS = 16
NTS = H // TS


def _search_body(q_ref, ka_ref, kb_ref, wts_ref, idx_ref):
    i = pl.program_id(0)
    q = q_ref[...]
    kc = jnp.concatenate([ka_ref[...], kb_ref[...]], axis=1)
    ds = []
    for o in range(81):
        dy, dx = o // 9 - 4, o % 9 - 4
        ks = kc[:, dy + 4:dy + 4 + TS, dx + 4:dx + 228]
        ds.append(jnp.sum(q * ks, axis=0))
    D = jnp.stack(ds, axis=0)
    lane0 = lax.broadcasted_iota(jnp.int32, (81, TS, W), 0)
    rem = D
    tds, tis = [], []
    for _ in range(K):
        m = jnp.max(rem, axis=0, keepdims=True)
        eq = rem == m
        first = jnp.min(jnp.where(eq, lane0, 81), axis=0, keepdims=True)
        tds.append(m[0])
        tis.append(first[0])
        rem = jnp.where(lane0 == first, -jnp.inf, rem)
    topd = jnp.stack(tds, axis=0)
    topi = jnp.stack(tis, axis=0)
    s = topd * 0.1
    e = jnp.exp(s - s[0:1])
    wts_ref[...] = e / jnp.sum(e, axis=0, keepdims=True)
    dyk = topi // 9 - 4
    dxk = topi % 9 - 4
    hloc = lax.broadcasted_iota(jnp.int32, (K, TS, W), 1) + i * TS
    wloc = lax.broadcasted_iota(jnp.int32, (K, TS, W), 2)
    gy = jnp.clip(hloc + dyk, 0, H - 1)
    gx = jnp.clip(wloc + dxk, 0, W - 1)
    idx_ref[...] = gy * W + gx


def _search(q3, kfp):
    return pl.pallas_call(
        _search_body,
        grid=(NTS,),
        in_specs=[
            pl.BlockSpec((32, TS, W), lambda i: (0, i, 0)),
            pl.BlockSpec((32, TS, W + 8), lambda i: (0, i, 0)),
            pl.BlockSpec((32, TS, W + 8), lambda i: (0, i + 1, 0)),
        ],
        out_specs=[
            pl.BlockSpec((K, TS, W), lambda i: (0, i, 0)),
            pl.BlockSpec((K, TS, W), lambda i: (0, i, 0)),
        ],
        out_shape=[
            jax.ShapeDtypeStruct((K, H, W), jnp.float32),
            jax.ShapeDtypeStruct((K, H, W), jnp.int32),
        ],
    )(q3, kfp, kfp)


_sc_mesh = plsc.VectorSubcoreMesh(core_axis_name="c", subcore_axis_name="s")


@functools.partial(
    pl.kernel,
    mesh=_sc_mesh,
    out_type=jax.ShapeDtypeStruct((K * HP * W, 128), jnp.float32),
    scratch_types=[
        pltpu.VMEM((W,), jnp.int32),
        pltpu.VMEM((W, 128), jnp.float32),
        pltpu.SemaphoreType.DMA,
    ],
)
def _sc_gather(x_hbm, idx_hbm, vol_hbm, idx_v, rows_v, sem):
    wid = lax.axis_index("s") * NC + lax.axis_index("c")

    def zfill(j, carry):
        for c in range(8):
            rows_v[j, pl.ds(c * 16, 16)] = jnp.zeros((16,), jnp.float32)
        return carry

    lax.fori_loop(0, W, zfill, 0)

    def zwrite(j, carry):
        z = wid + j * NW

        @pl.when(z < 8 * K)
        def _():
            k = z // 8
            r = z % 8
            row = jnp.where(r == 0, 0, H + r)
            pltpu.sync_copy(rows_v, vol_hbm.at[pl.ds((k * HP + row) * W, W)])

        return carry

    lax.fori_loop(0, 3, zwrite, 0)

    def body(j, carry):
        g = wid * PER_W + j
        k = g // H
        h = g % H
        pltpu.sync_copy(idx_hbm.at[pl.ds(g * W, W)], idx_v)
        pltpu.async_copy(x_hbm.at[idx_v], rows_v, sem).wait()
        pltpu.sync_copy(rows_v, vol_hbm.at[pl.ds((k * HP + h + 1) * W, W)])
        return carry

    lax.fori_loop(0, PER_W, body, 0)


def _conv_body(va_ref, vb_ref, wa_ref, wb_ref, wf_ref, bc_ref, out_ref):
    va = jnp.concatenate([va_ref[...], vb_ref[...]], axis=1)
    wc = jnp.concatenate([wa_ref[...], wb_ref[...]], axis=1)
    wvs = []
    for k in range(K):
        wk = wc[k][:, :, None]
        wvs.append((va[k, :, :, :C_IN] * wk).astype(jnp.bfloat16))
    bv = jnp.concatenate(wvs, axis=-1)
    zc = jnp.zeros((2 * TH, 1, K * C_IN), jnp.bfloat16)
    bvp = jnp.concatenate([zc, bv, zc], axis=1)
    acc = jnp.zeros((TH * W, C_OUT), jnp.float32) + bc_ref[...]
    for dy in range(3):
        for dx in range(3):
            patch = bvp[dy:dy + TH, dx:dx + W, :].reshape(TH * W, K * C_IN)
            acc = acc + jnp.dot(patch, wf_ref[dy * 3 + dx],
                                preferred_element_type=jnp.float32)
    out_ref[...] = acc.T.reshape(C_OUT, TH, W)


def _conv(vol, wtsp, Wf, bc):
    return pl.pallas_call(
        _conv_body,
        grid=(NT,),
        in_specs=[
            pl.BlockSpec((K, TH, W, 128), lambda i: (0, i, 0, 0)),
            pl.BlockSpec((K, TH, W, 128), lambda i: (0, i + 1, 0, 0)),
            pl.BlockSpec((K, TH, W), lambda i: (0, i, 0)),
            pl.BlockSpec((K, TH, W), lambda i: (0, i + 1, 0)),
            pl.BlockSpec((9, K * C_IN, C_OUT), lambda i: (0, 0, 0)),
            pl.BlockSpec((1, C_OUT), lambda i: (0, 0)),
        ],
        out_specs=pl.BlockSpec((C_OUT, TH, W), lambda i: (0, i, 0)),
        out_shape=jax.ShapeDtypeStruct((C_OUT, H, W), jnp.float32),
    )(vol, vol, wtsp, wtsp, Wf, bc)


def kernel(x, fflow, bflow, Wq, bq, Wk, bk, Wc, bc):
    x_cm = x[0].reshape(C_IN, HW)

    q_cm, kf_cm, x_pix128 = _proj(x_cm, Wq, bq.reshape(32, 1), Wk, bk.reshape(32, 1))
    q3 = q_cm.reshape(32, H, W)
    kfp = jnp.pad(kf_cm.reshape(32, H, W), ((0, 0), (4, 4), (4, 4)), mode="edge")
    kfp = jnp.pad(kfp, ((0, 0), (0, TS * (NTS + 1) - (H + 8)), (0, 0)))

    wts9, idx9 = _search(q3, kfp)

    vol = _sc_gather(x_pix128, idx9.reshape(-1))
    vol4 = vol.reshape(K, HP, W, 128)

    wtsp = jnp.pad(wts9, ((0, 0), (1, HP - H - 1), (0, 0)))
    Wf = jnp.transpose(Wc, (3, 4, 2, 1, 0)).reshape(
        9, K * C_IN, C_OUT).astype(jnp.bfloat16)
    out_cm = _conv(vol4, wtsp, Wf, bc.reshape(1, C_OUT))

    return out_cm[None]
